# baseline (speedup 1.0000x reference)
import functools

import jax
import jax.numpy as jnp
from jax import lax
from jax.experimental import pallas as pl
from jax.experimental.pallas import tpu as pltpu
from jax.experimental.pallas import tpu_sc as plsc

NN = 10000
NP = 10240
NE = 320000
DV = 72
DE = 14
DH = 128
EMB = 256
NM = 256
DEPTH_ITERS = 2

NC = 2
NS = 16
NW = NC * NS
EPW = NE // NW
CH = 40
NCHUNK = EPW // CH
ACC_ROWS_PER_TILE = NP // NS


def _sc_body(write_h, c_hbm, d_hbm, ei_hbm, *rest):
    if write_h:
        h_out, mv_out = rest[:2]
        rest = rest[2:]
    else:
        mv_out = rest[0]
        rest = rest[1:]
    (i0, i1, i2_, i3, i4_, i5, i6, i7, g0, g1, d0, d1, ob0, ob1, o0, o1, acc,
     gs0, gs1, ds0, ds1, ws0, ws1, ss0, ss1) = rest
    islot = (i0, i1, i2_, i3, i4_, i5, i6, i7)
    gbuf = (g0, g1)
    dbuf = (d0, d1)
    obf = (ob0, ob1)
    obuf = (o0, o1)
    gsem = (gs0, gs1)
    dsem = (ds0, ds1)
    wsem = (ws0, ws1)
    ssem = (ss0, ss1)
    cid = lax.axis_index("c")
    sid = lax.axis_index("s")
    w = cid * NS + sid
    e0 = w * EPW

    def zrow(r, _):
        for c8 in range(DH // 16):
            o0[r, pl.ds(c8 * 16, 16)] = jnp.zeros((16,), jnp.float32)
        return 0

    lax.fori_loop(0, CH, zrow, 0)
    for j in range(ACC_ROWS_PER_TILE // CH):
        pltpu.sync_copy(o0, acc.at[pl.ds(sid * ACC_ROWS_PER_TILE + j * CH, CH)])
    plsc.subcore_barrier()

    def issue_idx(j, k, b):
        pltpu.async_copy(ei_hbm.at[w, j], islot[k], dsem[b])

    def issue_inputs(j, k, b):
        pltpu.async_copy(c_hbm.at[islot[k].at[0]], gbuf[b], gsem[b])
        pltpu.async_copy(d_hbm.at[pl.ds(e0 + j * CH, CH)], dbuf[b], dsem[b])

    def issue_outputs(j, k, b):
        if write_h:
            pltpu.async_copy(obf[b], h_out.at[pl.ds(e0 + j * CH, CH)], wsem[b])
        pltpu.async_copy(obuf[b], acc.at[islot[k].at[1]], ssem[b], add=True)

    def wait_inputs(j, k, b, idx_slot):
        pltpu.make_async_copy(c_hbm.at[islot[k].at[0]], gbuf[b], gsem[b]).wait()
        pltpu.make_async_copy(d_hbm.at[pl.ds(e0 + j * CH, CH)], dbuf[b], dsem[b]).wait()
        if idx_slot is not None:
            pltpu.make_async_copy(ei_hbm.at[w, j], islot[idx_slot], dsem[b]).wait()

    def wait_outputs(j, k, b):
        if write_h:
            pltpu.make_async_copy(obf[b], h_out.at[pl.ds(e0 + j * CH, CH)], wsem[b]).wait()
        pltpu.make_async_copy(obuf[b], acc.at[islot[k].at[1]], ssem[b]).wait()

    def step(j, k, b, out_guard, idx_slot):
        def waits():
            wait_outputs(j - 2, k, b)
        if out_guard is None:
            waits()
        else:
            pl.when(out_guard)(waits)
        wait_inputs(j, k, b, idx_slot)

        def rowfn(r, _):
            for c4 in range(DH // 32):
                sl = pl.ds(c4 * 16, 16)
                clo = gbuf[b][r, pl.ds(c4 * 16, 16)]
                chi = gbuf[b][r, pl.ds(DHW + c4 * 16, 16)]
                di = dbuf[b][r, sl]
                dlo = lax.bitcast_convert_type(jnp.left_shift(di, 16), jnp.float32)
                dhi = lax.bitcast_convert_type(di & jnp.int32(-65536), jnp.float32)
                slo = jnp.maximum(clo + dlo, 0.0)
                shi = jnp.maximum(chi + dhi, 0.0)
                obuf[b][r, pl.ds(c4 * 32, 16)] = slo
                obuf[b][r, pl.ds(c4 * 32 + 16, 16)] = shi
                if write_h:
                    blo = lax.bitcast_convert_type(slo, jnp.int32)
                    bhi = lax.bitcast_convert_type(shi, jnp.int32)
                    wlo = lax.shift_right_logical(blo + jnp.int32(0x8000), 16)
                    whi = (bhi + jnp.int32(0x8000)) & jnp.int32(-65536)
                    obf[b][r, sl] = wlo | whi
            return 0

        lax.fori_loop(0, CH, rowfn, 0)
        issue_outputs(j, k, b)

    pltpu.sync_copy(ei_hbm.at[w, 0], islot[0])
    pltpu.sync_copy(ei_hbm.at[w, 1], islot[1])
    issue_inputs(0, 0, 0)
    issue_idx(2, 2, 0)
    issue_inputs(1, 1, 1)
    issue_idx(3, 3, 1)

    def octet(m, _):
        for k in range(8):
            j = m * 8 + k
            b = k % 2
            guard = (m >= 1) if k < 2 else None
            step(j, k, b, guard, (k + 2) % 8)
            issue_inputs(j + 2, (k + 2) % 8, b)

            def _issue_idx():
                issue_idx(j + 4, (k + 4) % 8, b)
            pl.when(j + 4 < NCHUNK)(_issue_idx)
        return 0

    lax.fori_loop(0, (NCHUNK - 2) // 8, octet, 0)
    for t in range(2):
        j = NCHUNK - 2 + t
        step(j, j % 8, j % 2, None, None)
    for t in range(2):
        j = NCHUNK - 2 + t
        wait_outputs(j, j % 8, j % 2)
    plsc.subcore_barrier()

    for j in range(ACC_ROWS_PER_TILE // CH):
        r0 = sid * ACC_ROWS_PER_TILE + j * CH
        pltpu.sync_copy(acc.at[pl.ds(r0, CH)], o0)
        pltpu.sync_copy(o0, mv_out.at[cid, pl.ds(r0, CH)])


@functools.cache
def _make_sc_fuse(write_h):
    mesh = plsc.VectorSubcoreMesh(core_axis_name="c", subcore_axis_name="s",
                                  num_cores=NC, num_subcores=NS)
    outs = []
    if write_h:
        outs.append(jax.ShapeDtypeStruct((NE, DH // 2), jnp.int32))
    outs.append(jax.ShapeDtypeStruct((NC, NP, DH), jnp.float32))
    return pl.kernel(
        functools.partial(_sc_body, write_h),
        out_type=tuple(outs) if write_h else outs[0],
        mesh=mesh,
        scratch_types=(
            [pltpu.VMEM((2, CH), jnp.int32) for _ in range(8)]
            + [pltpu.VMEM((CH, DH), jnp.float32) for _ in range(2)]
            + [pltpu.VMEM((CH, DH // 2), jnp.int32) for _ in range(4)]
            + [pltpu.VMEM((CH, DH), jnp.float32) for _ in range(2)]
            + [pltpu.VMEM_SHARED((NP, DH), jnp.float32)]
            + [pltpu.SemaphoreType.DMA for _ in range(8)]
        ),
    )


BR = 2560
BRE = 6400
BN = 1024
DHW = DH // 2


def _pack_bf16_pair(x):
    lo = lax.bitcast_convert_type(x[:, :DHW].astype(jnp.bfloat16), jnp.uint16)
    hi = lax.bitcast_convert_type(x[:, DHW:].astype(jnp.bfloat16), jnp.uint16)
    word = lo.astype(jnp.uint32) | (hi.astype(jnp.uint32) << 16)
    return lax.bitcast_convert_type(word, jnp.int32)


def _unpack_bf16_pair(u):
    w = lax.bitcast_convert_type(u, jnp.uint32)
    lo = lax.bitcast_convert_type((w & 0xFFFF).astype(jnp.uint16), jnp.bfloat16)
    hi = lax.bitcast_convert_type((w >> 16).astype(jnp.uint16), jnp.bfloat16)
    return jnp.concatenate([lo, hi], axis=1)


def _u_body(v_ref, w_ref, o_ref):
    o_ref[...] = jnp.dot(v_ref[...], w_ref[...], preferred_element_type=jnp.float32)


def _k_u(vp, wiv):
    return pl.pallas_call(
        _u_body,
        grid=(NP // BN,),
        in_specs=[
            pl.BlockSpec((BN, DV), lambda i: (i, 0)),
            pl.BlockSpec((DV, DH), lambda i: (0, 0)),
        ],
        out_specs=pl.BlockSpec((BN, DH), lambda i: (i, 0)),
        out_shape=jax.ShapeDtypeStruct((NP, DH), jnp.float32),
    )(vp, wiv)


def _ew_body(e_ref, w_ref, o_ref):
    o_ref[...] = _pack_bf16_pair(jnp.dot(e_ref[...], w_ref[...],
                                         preferred_element_type=jnp.float32))


def _k_ew(eattr, wie):
    return pl.pallas_call(
        _ew_body,
        grid=(NE // BRE,),
        in_specs=[
            pl.BlockSpec((BRE, DE), lambda i: (i, 0)),
            pl.BlockSpec((DE, DH), lambda i: (0, 0)),
        ],
        out_specs=pl.BlockSpec((BRE, DHW), lambda i: (i, 0)),
        out_shape=jax.ShapeDtypeStruct((NE, DHW), jnp.int32),
    )(eattr, wie)


def _edge_body(h_ref, e_ref, wh_ref, wie_ref, o_ref):
    h = _unpack_bf16_pair(h_ref[...])
    g = jnp.dot(h, wh_ref[...].astype(jnp.bfloat16),
                preferred_element_type=jnp.float32)
    up = jnp.concatenate([g[1:], g[:1]], axis=0)
    down = jnp.concatenate([g[-1:], g[:-1]], axis=0)
    row = lax.broadcasted_iota(jnp.int32, (BR, DH), 0)
    sw = jnp.where((row % 2) == 0, up, down)
    ew = jnp.dot(e_ref[...], wie_ref[...], preferred_element_type=jnp.float32)
    o_ref[...] = _pack_bf16_pair(ew - sw)


def _k_edge(h, eattr, wh, wie):
    return pl.pallas_call(
        _edge_body,
        grid=(NE // BR,),
        in_specs=[
            pl.BlockSpec((BR, DHW), lambda i: (i, 0)),
            pl.BlockSpec((BR, DE), lambda i: (i, 0)),
            pl.BlockSpec((DH, DH), lambda i: (0, 0)),
            pl.BlockSpec((DE, DH), lambda i: (0, 0)),
        ],
        out_specs=pl.BlockSpec((BR, DHW), lambda i: (i, 0)),
        out_shape=jax.ShapeDtypeStruct((NE, DHW), jnp.int32),
    )(h, eattr, wh, wie)


def _table_body(p_ref, u_ref, wh_ref, o_ref):
    mv = p_ref[0] + p_ref[1]
    o_ref[...] = u_ref[...] + jnp.dot(mv, wh_ref[...],
                                      preferred_element_type=jnp.float32)


def _k_table(p, u, wh):
    return pl.pallas_call(
        _table_body,
        grid=(NP // BN,),
        in_specs=[
            pl.BlockSpec((NC, BN, DH), lambda i: (0, i, 0)),
            pl.BlockSpec((BN, DH), lambda i: (i, 0)),
            pl.BlockSpec((DH, DH), lambda i: (0, 0)),
        ],
        out_specs=pl.BlockSpec((BN, DH), lambda i: (i, 0)),
        out_shape=jax.ShapeDtypeStruct((NP, DH), jnp.float32),
    )(p, u, wh)


def _final_body(p_ref, v_ref, b_ref, wov_ref, wom_ref, bo_ref, w1_ref, b1_ref,
                w2_ref, b2_ref, o_ref, zs_acc, cnt_acc):
    i = pl.program_id(0)

    @pl.when(i == 0)
    def _():
        zs_acc[...] = jnp.zeros_like(zs_acc)
        cnt_acc[...] = jnp.zeros_like(cnt_acc)

    m = p_ref[0] + p_ref[1]
    hv = jnp.maximum(
        jnp.dot(v_ref[...], wov_ref[...], preferred_element_type=jnp.float32)
        + jnp.dot(m, wom_ref[...], preferred_element_type=jnp.float32)
        + bo_ref[...],
        0.0,
    )
    blab = b_ref[...].reshape(1, BN)
    oht = (lax.broadcasted_iota(jnp.int32, (NM, BN), 0) == blab).astype(jnp.float32)
    zs_acc[...] += jnp.dot(oht, hv, preferred_element_type=jnp.float32)
    cnt_acc[...] += jnp.dot(oht, jnp.ones((BN, DH), jnp.float32),
                            preferred_element_type=jnp.float32)

    @pl.when(i == NP // BN - 1)
    def _():
        z = zs_acc[...] / jnp.maximum(cnt_acc[...], 1.0)
        h = jnp.maximum(
            jnp.dot(z, w1_ref[...], preferred_element_type=jnp.float32) + b1_ref[...],
            0.0,
        )
        o_ref[...] = jnp.dot(h, w2_ref[...], preferred_element_type=jnp.float32) + b2_ref[...]


def _k_final(p, vp, batch3, wov, wom, bo2, w1, b12, w2, b22):
    return pl.pallas_call(
        _final_body,
        grid=(NP // BN,),
        in_specs=[
            pl.BlockSpec((NC, BN, DH), lambda i: (0, i, 0)),
            pl.BlockSpec((BN, DV), lambda i: (i, 0)),
            pl.BlockSpec((1, 1, BN), lambda i: (i, 0, 0)),
            pl.BlockSpec((DV, DH), lambda i: (0, 0)),
            pl.BlockSpec((DH, DH), lambda i: (0, 0)),
            pl.BlockSpec((1, DH), lambda i: (0, 0)),
            pl.BlockSpec((DH, EMB), lambda i: (0, 0)),
            pl.BlockSpec((1, EMB), lambda i: (0, 0)),
            pl.BlockSpec((EMB, EMB), lambda i: (0, 0)),
            pl.BlockSpec((1, EMB), lambda i: (0, 0)),
        ],
        out_specs=pl.BlockSpec((NM, EMB), lambda i: (0, 0)),
        out_shape=jax.ShapeDtypeStruct((NM, EMB), jnp.float32),
        scratch_shapes=[
            pltpu.VMEM((NM, DH), jnp.float32),
            pltpu.VMEM((NM, DH), jnp.float32),
        ],
    )(p, vp, batch3, wov, wom, bo2, w1, b12, w2, b22)


_SIGMA = sum(([16 * c + i for i in range(16)]
              + [64 + 16 * c + i for i in range(16)]
              for c in range(DH // 32)), [])


def kernel(V, Eattr, edge_index, rev_edge_index, batch, Wi, Wh, Wo, bo, W1, b1, W2, b2):
    ei3 = edge_index.reshape(2, NW, NCHUNK, CH).transpose(1, 2, 0, 3)
    wiv, wie = Wi[:DV], Wi[DV:]
    wov, wom = Wo[:DV], Wo[DV:]
    sigma = jnp.array(_SIGMA, dtype=jnp.int32)
    wh_sig = Wh[sigma]
    wom_sig = wom[sigma]
    vp = jnp.pad(V, ((0, NP - NN), (0, 0)))
    batch3 = jnp.pad(batch, (0, NP - NN), constant_values=NM).reshape(NP // BN, 1, BN)

    u = _k_u(vp, wiv)
    ew = _k_ew(Eattr, wie)
    h, p = _make_sc_fuse(True)(u, ew, ei3)
    for it in range(DEPTH_ITERS):
        c = _k_table(p, u, wh_sig)
        d = _k_edge(h, Eattr, Wh, wie)
        if it < DEPTH_ITERS - 1:
            h, p = _make_sc_fuse(True)(c, d, ei3)
        else:
            p = _make_sc_fuse(False)(c, d, ei3)
    return _k_final(p, vp, batch3, wov, wom_sig, bo.reshape(1, DH), W1,
                    b1.reshape(1, EMB), W2, b2.reshape(1, EMB))


# speedup vs baseline: 1.0134x; 1.0134x over previous
import functools

import jax
import jax.numpy as jnp
from jax import lax
from jax.experimental import pallas as pl
from jax.experimental.pallas import tpu as pltpu
from jax.experimental.pallas import tpu_sc as plsc

NN = 10000
NP = 10240
NE = 320000
DV = 72
DE = 14
DH = 128
EMB = 256
NM = 256
DEPTH_ITERS = 2

NC = 2
NS = 16
NW = NC * NS
EPW = NE // NW
CH = 40
NCHUNK = EPW // CH
ACC_ROWS_PER_TILE = NP // NS


def _sc_body(write_h, c_hbm, d_hbm, ei_hbm, *rest):
    if write_h:
        h_out, mv_out = rest[:2]
        rest = rest[2:]
    else:
        mv_out = rest[0]
        rest = rest[1:]
    (i0, i1, i2_, i3, i4_, i5, i6, i7, g0, g1, d0, d1, o0, o1, acc,
     gs0, gs1, ds0, ds1, ws0, ws1, ss0, ss1) = rest
    islot = (i0, i1, i2_, i3, i4_, i5, i6, i7)
    gbuf = (g0, g1)
    dbuf = (d0, d1)
    obuf = (o0, o1)
    gsem = (gs0, gs1)
    dsem = (ds0, ds1)
    wsem = (ws0, ws1)
    ssem = (ss0, ss1)
    cid = lax.axis_index("c")
    sid = lax.axis_index("s")
    w = cid * NS + sid
    e0 = w * EPW

    def zrow(r, _):
        for c8 in range(DH // 16):
            o0[r, pl.ds(c8 * 16, 16)] = jnp.zeros((16,), jnp.float32)
        return 0

    lax.fori_loop(0, CH, zrow, 0)
    for j in range(ACC_ROWS_PER_TILE // CH):
        pltpu.sync_copy(o0, acc.at[pl.ds(sid * ACC_ROWS_PER_TILE + j * CH, CH)])
    plsc.subcore_barrier()

    def issue_idx(j, k, b):
        pltpu.async_copy(ei_hbm.at[w, j], islot[k], dsem[b])

    def issue_inputs(j, k, b):
        pltpu.async_copy(c_hbm.at[islot[k].at[0]], gbuf[b], gsem[b])
        pltpu.async_copy(d_hbm.at[pl.ds(e0 + j * CH, CH)], dbuf[b], dsem[b])

    def issue_outputs(j, k, b):
        if write_h:
            pltpu.async_copy(obuf[b], h_out.at[pl.ds(e0 + j * CH, CH)], wsem[b])
        pltpu.async_copy(obuf[b], acc.at[islot[k].at[1]], ssem[b], add=True)

    def wait_inputs(j, k, b, idx_slot):
        pltpu.make_async_copy(c_hbm.at[islot[k].at[0]], gbuf[b], gsem[b]).wait()
        pltpu.make_async_copy(d_hbm.at[pl.ds(e0 + j * CH, CH)], dbuf[b], dsem[b]).wait()
        if idx_slot is not None:
            pltpu.make_async_copy(ei_hbm.at[w, j], islot[idx_slot], dsem[b]).wait()

    def wait_outputs(j, k, b):
        if write_h:
            pltpu.make_async_copy(obuf[b], h_out.at[pl.ds(e0 + j * CH, CH)], wsem[b]).wait()
        pltpu.make_async_copy(obuf[b], acc.at[islot[k].at[1]], ssem[b]).wait()

    def step(j, k, b, out_guard, idx_slot):
        def waits():
            wait_outputs(j - 2, k, b)
        if out_guard is None:
            waits()
        else:
            pl.when(out_guard)(waits)
        wait_inputs(j, k, b, idx_slot)

        def rowfn(r, _):
            for c4 in range(DH // 32):
                lo = pl.ds(c4 * 16, 16)
                hi = pl.ds(DHW + c4 * 16, 16)
                di = dbuf[b][r, lo]
                dlo = lax.bitcast_convert_type(jnp.left_shift(di, 16), jnp.float32)
                dhi = lax.bitcast_convert_type(di & jnp.int32(-65536), jnp.float32)
                obuf[b][r, lo] = jnp.maximum(gbuf[b][r, lo] + dlo, 0.0)
                obuf[b][r, hi] = jnp.maximum(gbuf[b][r, hi] + dhi, 0.0)
            return 0

        lax.fori_loop(0, CH, rowfn, 0)
        issue_outputs(j, k, b)

    pltpu.sync_copy(ei_hbm.at[w, 0], islot[0])
    pltpu.sync_copy(ei_hbm.at[w, 1], islot[1])
    issue_inputs(0, 0, 0)
    issue_idx(2, 2, 0)
    issue_inputs(1, 1, 1)
    issue_idx(3, 3, 1)

    def octet(m, _):
        for k in range(8):
            j = m * 8 + k
            b = k % 2
            guard = (m >= 1) if k < 2 else None
            step(j, k, b, guard, (k + 2) % 8)
            issue_inputs(j + 2, (k + 2) % 8, b)

            def _issue_idx():
                issue_idx(j + 4, (k + 4) % 8, b)
            pl.when(j + 4 < NCHUNK)(_issue_idx)
        return 0

    lax.fori_loop(0, (NCHUNK - 2) // 8, octet, 0)
    for t in range(2):
        j = NCHUNK - 2 + t
        step(j, j % 8, j % 2, None, None)
    for t in range(2):
        j = NCHUNK - 2 + t
        wait_outputs(j, j % 8, j % 2)
    plsc.subcore_barrier()

    for j in range(ACC_ROWS_PER_TILE // CH):
        r0 = sid * ACC_ROWS_PER_TILE + j * CH
        pltpu.sync_copy(acc.at[pl.ds(r0, CH)], o0)
        pltpu.sync_copy(o0, mv_out.at[cid, pl.ds(r0, CH)])


@functools.cache
def _make_sc_fuse(write_h):
    mesh = plsc.VectorSubcoreMesh(core_axis_name="c", subcore_axis_name="s",
                                  num_cores=NC, num_subcores=NS)
    outs = []
    if write_h:
        outs.append(jax.ShapeDtypeStruct((NE, DH), jnp.float32))
    outs.append(jax.ShapeDtypeStruct((NC, NP, DH), jnp.float32))
    return pl.kernel(
        functools.partial(_sc_body, write_h),
        out_type=tuple(outs) if write_h else outs[0],
        mesh=mesh,
        scratch_types=(
            [pltpu.VMEM((2, CH), jnp.int32) for _ in range(8)]
            + [pltpu.VMEM((CH, DH), jnp.float32) for _ in range(2)]
            + [pltpu.VMEM((CH, DH // 2), jnp.int32) for _ in range(2)]
            + [pltpu.VMEM((CH, DH), jnp.float32) for _ in range(2)]
            + [pltpu.VMEM_SHARED((NP, DH), jnp.float32)]
            + [pltpu.SemaphoreType.DMA for _ in range(8)]
        ),
    )


BR = 2560
BRE = 6400
BN = 1024
DHW = DH // 2


def _pack_bf16_pair(x):
    lo = lax.bitcast_convert_type(x[:, :DHW].astype(jnp.bfloat16), jnp.uint16)
    hi = lax.bitcast_convert_type(x[:, DHW:].astype(jnp.bfloat16), jnp.uint16)
    word = lo.astype(jnp.uint32) | (hi.astype(jnp.uint32) << 16)
    return lax.bitcast_convert_type(word, jnp.int32)


def _unpack_bf16_pair(u):
    w = lax.bitcast_convert_type(u, jnp.uint32)
    lo = lax.bitcast_convert_type((w & 0xFFFF).astype(jnp.uint16), jnp.bfloat16)
    hi = lax.bitcast_convert_type((w >> 16).astype(jnp.uint16), jnp.bfloat16)
    return jnp.concatenate([lo, hi], axis=1)


def _u_body(v_ref, w_ref, o_ref):
    o_ref[...] = jnp.dot(v_ref[...], w_ref[...], preferred_element_type=jnp.float32)


def _k_u(vp, wiv):
    return pl.pallas_call(
        _u_body,
        grid=(NP // BN,),
        in_specs=[
            pl.BlockSpec((BN, DV), lambda i: (i, 0)),
            pl.BlockSpec((DV, DH), lambda i: (0, 0)),
        ],
        out_specs=pl.BlockSpec((BN, DH), lambda i: (i, 0)),
        out_shape=jax.ShapeDtypeStruct((NP, DH), jnp.float32),
    )(vp, wiv)


def _ew_body(e_ref, w_ref, o_ref):
    o_ref[...] = _pack_bf16_pair(jnp.dot(e_ref[...], w_ref[...],
                                         preferred_element_type=jnp.float32))


def _k_ew(eattr, wie):
    return pl.pallas_call(
        _ew_body,
        grid=(NE // BRE,),
        in_specs=[
            pl.BlockSpec((BRE, DE), lambda i: (i, 0)),
            pl.BlockSpec((DE, DH), lambda i: (0, 0)),
        ],
        out_specs=pl.BlockSpec((BRE, DHW), lambda i: (i, 0)),
        out_shape=jax.ShapeDtypeStruct((NE, DHW), jnp.int32),
    )(eattr, wie)


def _edge_body(h_ref, e_ref, wh_ref, wie_ref, o_ref):
    g = jnp.dot(h_ref[...].astype(jnp.bfloat16), wh_ref[...].astype(jnp.bfloat16),
                preferred_element_type=jnp.float32)
    up = jnp.concatenate([g[1:], g[:1]], axis=0)
    down = jnp.concatenate([g[-1:], g[:-1]], axis=0)
    row = lax.broadcasted_iota(jnp.int32, (BR, DH), 0)
    sw = jnp.where((row % 2) == 0, up, down)
    ew = jnp.dot(e_ref[...], wie_ref[...], preferred_element_type=jnp.float32)
    o_ref[...] = _pack_bf16_pair(ew - sw)


def _k_edge(h, eattr, wh, wie):
    return pl.pallas_call(
        _edge_body,
        grid=(NE // BR,),
        in_specs=[
            pl.BlockSpec((BR, DH), lambda i: (i, 0)),
            pl.BlockSpec((BR, DE), lambda i: (i, 0)),
            pl.BlockSpec((DH, DH), lambda i: (0, 0)),
            pl.BlockSpec((DE, DH), lambda i: (0, 0)),
        ],
        out_specs=pl.BlockSpec((BR, DHW), lambda i: (i, 0)),
        out_shape=jax.ShapeDtypeStruct((NE, DHW), jnp.int32),
    )(h, eattr, wh, wie)


def _table_body(p_ref, u_ref, wh_ref, o_ref):
    mv = p_ref[0] + p_ref[1]
    o_ref[...] = u_ref[...] + jnp.dot(mv, wh_ref[...],
                                      preferred_element_type=jnp.float32)


def _k_table(p, u, wh):
    return pl.pallas_call(
        _table_body,
        grid=(NP // BN,),
        in_specs=[
            pl.BlockSpec((NC, BN, DH), lambda i: (0, i, 0)),
            pl.BlockSpec((BN, DH), lambda i: (i, 0)),
            pl.BlockSpec((DH, DH), lambda i: (0, 0)),
        ],
        out_specs=pl.BlockSpec((BN, DH), lambda i: (i, 0)),
        out_shape=jax.ShapeDtypeStruct((NP, DH), jnp.float32),
    )(p, u, wh)


def _final_body(p_ref, v_ref, b_ref, wov_ref, wom_ref, bo_ref, w1_ref, b1_ref,
                w2_ref, b2_ref, o_ref, zs_acc, cnt_acc):
    i = pl.program_id(0)

    @pl.when(i == 0)
    def _():
        zs_acc[...] = jnp.zeros_like(zs_acc)
        cnt_acc[...] = jnp.zeros_like(cnt_acc)

    m = p_ref[0] + p_ref[1]
    hv = jnp.maximum(
        jnp.dot(v_ref[...], wov_ref[...], preferred_element_type=jnp.float32)
        + jnp.dot(m, wom_ref[...], preferred_element_type=jnp.float32)
        + bo_ref[...],
        0.0,
    )
    blab = b_ref[...].reshape(1, BN)
    oht = (lax.broadcasted_iota(jnp.int32, (NM, BN), 0) == blab).astype(jnp.float32)
    zs_acc[...] += jnp.dot(oht, hv, preferred_element_type=jnp.float32)
    cnt_acc[...] += jnp.dot(oht, jnp.ones((BN, DH), jnp.float32),
                            preferred_element_type=jnp.float32)

    @pl.when(i == NP // BN - 1)
    def _():
        z = zs_acc[...] / jnp.maximum(cnt_acc[...], 1.0)
        h = jnp.maximum(
            jnp.dot(z, w1_ref[...], preferred_element_type=jnp.float32) + b1_ref[...],
            0.0,
        )
        o_ref[...] = jnp.dot(h, w2_ref[...], preferred_element_type=jnp.float32) + b2_ref[...]


def _k_final(p, vp, batch3, wov, wom, bo2, w1, b12, w2, b22):
    return pl.pallas_call(
        _final_body,
        grid=(NP // BN,),
        in_specs=[
            pl.BlockSpec((NC, BN, DH), lambda i: (0, i, 0)),
            pl.BlockSpec((BN, DV), lambda i: (i, 0)),
            pl.BlockSpec((1, 1, BN), lambda i: (i, 0, 0)),
            pl.BlockSpec((DV, DH), lambda i: (0, 0)),
            pl.BlockSpec((DH, DH), lambda i: (0, 0)),
            pl.BlockSpec((1, DH), lambda i: (0, 0)),
            pl.BlockSpec((DH, EMB), lambda i: (0, 0)),
            pl.BlockSpec((1, EMB), lambda i: (0, 0)),
            pl.BlockSpec((EMB, EMB), lambda i: (0, 0)),
            pl.BlockSpec((1, EMB), lambda i: (0, 0)),
        ],
        out_specs=pl.BlockSpec((NM, EMB), lambda i: (0, 0)),
        out_shape=jax.ShapeDtypeStruct((NM, EMB), jnp.float32),
        scratch_shapes=[
            pltpu.VMEM((NM, DH), jnp.float32),
            pltpu.VMEM((NM, DH), jnp.float32),
        ],
    )(p, vp, batch3, wov, wom, bo2, w1, b12, w2, b22)


def kernel(V, Eattr, edge_index, rev_edge_index, batch, Wi, Wh, Wo, bo, W1, b1, W2, b2):
    ei3 = edge_index.reshape(2, NW, NCHUNK, CH).transpose(1, 2, 0, 3)
    wiv, wie = Wi[:DV], Wi[DV:]
    wov, wom = Wo[:DV], Wo[DV:]
    vp = jnp.pad(V, ((0, NP - NN), (0, 0)))
    batch3 = jnp.pad(batch, (0, NP - NN), constant_values=NM).reshape(NP // BN, 1, BN)

    u = _k_u(vp, wiv)
    ew = _k_ew(Eattr, wie)
    h, p = _make_sc_fuse(True)(u, ew, ei3)
    for it in range(DEPTH_ITERS):
        c = _k_table(p, u, Wh)
        d = _k_edge(h, Eattr, Wh, wie)
        if it < DEPTH_ITERS - 1:
            h, p = _make_sc_fuse(True)(c, d, ei3)
        else:
            p = _make_sc_fuse(False)(c, d, ei3)
    return _k_final(p, vp, batch3, wov, wom, bo.reshape(1, DH), W1,
                    b1.reshape(1, EMB), W2, b2.reshape(1, EMB))


# speedup vs baseline: 1.0397x; 1.0259x over previous
import functools

import jax
import jax.numpy as jnp
from jax import lax
from jax.experimental import pallas as pl
from jax.experimental.pallas import tpu as pltpu
from jax.experimental.pallas import tpu_sc as plsc

NN = 10000
NP = 10240
NE = 320000
DV = 72
DE = 14
DH = 128
EMB = 256
NM = 256
DEPTH_ITERS = 2

NC = 2
NS = 16
NW = NC * NS
EPW = NE // NW
CH = 40
NCHUNK = EPW // CH
ACC_ROWS_PER_TILE = NP // NS


def _sc_body(write_h, c_hbm, d_hbm, ei_hbm, *rest):
    if write_h:
        h_out, mv_out = rest[:2]
        rest = rest[2:]
    else:
        mv_out = rest[0]
        rest = rest[1:]
    (i0, i1, i2_, i3, i4_, i5, i6, i7, g0, g1, d0, d1, o0, o1, acc,
     gs0, gs1, ds0, ds1, ws0, ws1, ss0, ss1) = rest
    islot = (i0, i1, i2_, i3, i4_, i5, i6, i7)
    gbuf = (g0, g1)
    dbuf = (d0, d1)
    obuf = (o0, o1)
    gsem = (gs0, gs1)
    dsem = (ds0, ds1)
    wsem = (ws0, ws1)
    ssem = (ss0, ss1)
    cid = lax.axis_index("c")
    sid = lax.axis_index("s")
    w = cid * NS + sid
    e0 = w * EPW

    def zrow(r, _):
        for c8 in range(DH // 16):
            o0[r, pl.ds(c8 * 16, 16)] = jnp.zeros((16,), jnp.float32)
        return 0

    lax.fori_loop(0, CH, zrow, 0)
    for j in range(ACC_ROWS_PER_TILE // CH):
        pltpu.sync_copy(o0, acc.at[pl.ds(sid * ACC_ROWS_PER_TILE + j * CH, CH)])
    plsc.subcore_barrier()

    def issue_idx(j, k, b):
        pltpu.async_copy(ei_hbm.at[w, j], islot[k], dsem[b])

    def issue_inputs(j, k, b):
        pltpu.async_copy(c_hbm.at[islot[k].at[0]], gbuf[b], gsem[b])
        pltpu.async_copy(d_hbm.at[pl.ds(e0 + j * CH, CH)], dbuf[b], dsem[b])

    def issue_outputs(j, k, b):
        if write_h:
            pltpu.async_copy(obuf[b], h_out.at[pl.ds(e0 + j * CH, CH)], wsem[b])
        pltpu.async_copy(obuf[b], acc.at[islot[k].at[1]], ssem[b], add=True)

    def wait_inputs(j, k, b, idx_slot):
        pltpu.make_async_copy(c_hbm.at[islot[k].at[0]], gbuf[b], gsem[b]).wait()
        pltpu.make_async_copy(d_hbm.at[pl.ds(e0 + j * CH, CH)], dbuf[b], dsem[b]).wait()
        if idx_slot is not None:
            pltpu.make_async_copy(ei_hbm.at[w, j], islot[idx_slot], dsem[b]).wait()

    def wait_outputs(j, k, b):
        if write_h:
            pltpu.make_async_copy(obuf[b], h_out.at[pl.ds(e0 + j * CH, CH)], wsem[b]).wait()
        pltpu.make_async_copy(obuf[b], acc.at[islot[k].at[1]], ssem[b]).wait()

    def step(j, k, b, out_guard, idx_slot):
        def waits():
            wait_outputs(j - 2, k, b)
        if out_guard is None:
            waits()
        else:
            pl.when(out_guard)(waits)
        wait_inputs(j, k, b, idx_slot)

        def rowfn(r, _):
            for c8 in range(DH // 16):
                sl = pl.ds(c8 * 16, 16)
                obuf[b][r, sl] = jnp.maximum(gbuf[b][r, sl] + dbuf[b][r, sl], 0.0)
            return 0

        lax.fori_loop(0, CH, rowfn, 0)
        issue_outputs(j, k, b)

    pltpu.sync_copy(ei_hbm.at[w, 0], islot[0])
    pltpu.sync_copy(ei_hbm.at[w, 1], islot[1])
    issue_inputs(0, 0, 0)
    issue_idx(2, 2, 0)
    issue_inputs(1, 1, 1)
    issue_idx(3, 3, 1)

    def octet(m, _):
        for k in range(8):
            j = m * 8 + k
            b = k % 2
            guard = (m >= 1) if k < 2 else None
            step(j, k, b, guard, (k + 2) % 8)
            issue_inputs(j + 2, (k + 2) % 8, b)

            def _issue_idx():
                issue_idx(j + 4, (k + 4) % 8, b)
            pl.when(j + 4 < NCHUNK)(_issue_idx)
        return 0

    lax.fori_loop(0, (NCHUNK - 2) // 8, octet, 0)
    for t in range(2):
        j = NCHUNK - 2 + t
        step(j, j % 8, j % 2, None, None)
    for t in range(2):
        j = NCHUNK - 2 + t
        wait_outputs(j, j % 8, j % 2)
    plsc.subcore_barrier()

    for j in range(ACC_ROWS_PER_TILE // CH):
        r0 = sid * ACC_ROWS_PER_TILE + j * CH
        pltpu.sync_copy(acc.at[pl.ds(r0, CH)], o0)
        pltpu.sync_copy(o0, mv_out.at[cid, pl.ds(r0, CH)])


@functools.cache
def _make_sc_fuse(write_h):
    mesh = plsc.VectorSubcoreMesh(core_axis_name="c", subcore_axis_name="s",
                                  num_cores=NC, num_subcores=NS)
    outs = []
    if write_h:
        outs.append(jax.ShapeDtypeStruct((NE, DH), jnp.float32))
    outs.append(jax.ShapeDtypeStruct((NC, NP, DH), jnp.float32))
    return pl.kernel(
        functools.partial(_sc_body, write_h),
        out_type=tuple(outs) if write_h else outs[0],
        mesh=mesh,
        scratch_types=(
            [pltpu.VMEM((2, CH), jnp.int32) for _ in range(8)]
            + [pltpu.VMEM((CH, DH), jnp.float32) for _ in range(6)]
            + [pltpu.VMEM_SHARED((NP, DH), jnp.float32)]
            + [pltpu.SemaphoreType.DMA for _ in range(8)]
        ),
    )


BR = 2560
BRE = 6400
BN = 1024


def _u_body(v_ref, w_ref, o_ref):
    o_ref[...] = jnp.dot(v_ref[...], w_ref[...], preferred_element_type=jnp.float32)


def _k_u(vp, wiv):
    return pl.pallas_call(
        _u_body,
        grid=(NP // BN,),
        in_specs=[
            pl.BlockSpec((BN, DV), lambda i: (i, 0)),
            pl.BlockSpec((DV, DH), lambda i: (0, 0)),
        ],
        out_specs=pl.BlockSpec((BN, DH), lambda i: (i, 0)),
        out_shape=jax.ShapeDtypeStruct((NP, DH), jnp.float32),
    )(vp, wiv)


def _ew_body(e_ref, w_ref, o_ref):
    o_ref[...] = jnp.dot(e_ref[...], w_ref[...], preferred_element_type=jnp.float32)


def _k_ew(eattr, wie):
    return pl.pallas_call(
        _ew_body,
        grid=(NE // BRE,),
        in_specs=[
            pl.BlockSpec((BRE, DE), lambda i: (i, 0)),
            pl.BlockSpec((DE, DH), lambda i: (0, 0)),
        ],
        out_specs=pl.BlockSpec((BRE, DH), lambda i: (i, 0)),
        out_shape=jax.ShapeDtypeStruct((NE, DH), jnp.float32),
    )(eattr, wie)


def _edge_body(h_ref, e_ref, wh_ref, wie_ref, o_ref):
    g = jnp.dot(h_ref[...], wh_ref[...], preferred_element_type=jnp.float32)
    up = jnp.concatenate([g[1:], g[:1]], axis=0)
    down = jnp.concatenate([g[-1:], g[:-1]], axis=0)
    row = lax.broadcasted_iota(jnp.int32, (BR, DH), 0)
    sw = jnp.where((row % 2) == 0, up, down)
    ew = jnp.dot(e_ref[...], wie_ref[...], preferred_element_type=jnp.float32)
    o_ref[...] = ew - sw


def _k_edge(h, eattr, wh, wie):
    return pl.pallas_call(
        _edge_body,
        grid=(NE // BR,),
        in_specs=[
            pl.BlockSpec((BR, DH), lambda i: (i, 0)),
            pl.BlockSpec((BR, DE), lambda i: (i, 0)),
            pl.BlockSpec((DH, DH), lambda i: (0, 0)),
            pl.BlockSpec((DE, DH), lambda i: (0, 0)),
        ],
        out_specs=pl.BlockSpec((BR, DH), lambda i: (i, 0)),
        out_shape=jax.ShapeDtypeStruct((NE, DH), jnp.float32),
    )(h, eattr, wh, wie)


def _table_body(p_ref, u_ref, wh_ref, o_ref):
    mv = p_ref[0] + p_ref[1]
    o_ref[...] = u_ref[...] + jnp.dot(mv, wh_ref[...],
                                      preferred_element_type=jnp.float32)


def _k_table(p, u, wh):
    return pl.pallas_call(
        _table_body,
        grid=(NP // BN,),
        in_specs=[
            pl.BlockSpec((NC, BN, DH), lambda i: (0, i, 0)),
            pl.BlockSpec((BN, DH), lambda i: (i, 0)),
            pl.BlockSpec((DH, DH), lambda i: (0, 0)),
        ],
        out_specs=pl.BlockSpec((BN, DH), lambda i: (i, 0)),
        out_shape=jax.ShapeDtypeStruct((NP, DH), jnp.float32),
    )(p, u, wh)


def _final_body(p_ref, v_ref, b_ref, wov_ref, wom_ref, bo_ref, w1_ref, b1_ref,
                w2_ref, b2_ref, o_ref, zs_acc, cnt_acc):
    i = pl.program_id(0)

    @pl.when(i == 0)
    def _():
        zs_acc[...] = jnp.zeros_like(zs_acc)
        cnt_acc[...] = jnp.zeros_like(cnt_acc)

    m = p_ref[0] + p_ref[1]
    hv = jnp.maximum(
        jnp.dot(v_ref[...], wov_ref[...], preferred_element_type=jnp.float32)
        + jnp.dot(m, wom_ref[...], preferred_element_type=jnp.float32)
        + bo_ref[...],
        0.0,
    )
    blab = b_ref[...].reshape(1, BN)
    oht = (lax.broadcasted_iota(jnp.int32, (NM, BN), 0) == blab).astype(jnp.float32)
    zs_acc[...] += jnp.dot(oht, hv, preferred_element_type=jnp.float32)
    cnt_acc[...] += jnp.dot(oht, jnp.ones((BN, DH), jnp.float32),
                            preferred_element_type=jnp.float32)

    @pl.when(i == NP // BN - 1)
    def _():
        z = zs_acc[...] / jnp.maximum(cnt_acc[...], 1.0)
        h = jnp.maximum(
            jnp.dot(z, w1_ref[...], preferred_element_type=jnp.float32) + b1_ref[...],
            0.0,
        )
        o_ref[...] = jnp.dot(h, w2_ref[...], preferred_element_type=jnp.float32) + b2_ref[...]


def _k_final(p, vp, batch3, wov, wom, bo2, w1, b12, w2, b22):
    return pl.pallas_call(
        _final_body,
        grid=(NP // BN,),
        in_specs=[
            pl.BlockSpec((NC, BN, DH), lambda i: (0, i, 0)),
            pl.BlockSpec((BN, DV), lambda i: (i, 0)),
            pl.BlockSpec((1, 1, BN), lambda i: (i, 0, 0)),
            pl.BlockSpec((DV, DH), lambda i: (0, 0)),
            pl.BlockSpec((DH, DH), lambda i: (0, 0)),
            pl.BlockSpec((1, DH), lambda i: (0, 0)),
            pl.BlockSpec((DH, EMB), lambda i: (0, 0)),
            pl.BlockSpec((1, EMB), lambda i: (0, 0)),
            pl.BlockSpec((EMB, EMB), lambda i: (0, 0)),
            pl.BlockSpec((1, EMB), lambda i: (0, 0)),
        ],
        out_specs=pl.BlockSpec((NM, EMB), lambda i: (0, 0)),
        out_shape=jax.ShapeDtypeStruct((NM, EMB), jnp.float32),
        scratch_shapes=[
            pltpu.VMEM((NM, DH), jnp.float32),
            pltpu.VMEM((NM, DH), jnp.float32),
        ],
    )(p, vp, batch3, wov, wom, bo2, w1, b12, w2, b22)


def kernel(V, Eattr, edge_index, rev_edge_index, batch, Wi, Wh, Wo, bo, W1, b1, W2, b2):
    ei3 = edge_index.reshape(2, NW, NCHUNK, CH).transpose(1, 2, 0, 3)
    wiv, wie = Wi[:DV], Wi[DV:]
    wov, wom = Wo[:DV], Wo[DV:]
    vp = jnp.pad(V, ((0, NP - NN), (0, 0)))
    batch3 = jnp.pad(batch, (0, NP - NN), constant_values=NM).reshape(NP // BN, 1, BN)

    u = _k_u(vp, wiv)
    ew = _k_ew(Eattr, wie)
    h, p = _make_sc_fuse(True)(u, ew, ei3)
    for it in range(DEPTH_ITERS):
        c = _k_table(p, u, Wh)
        d = _k_edge(h, Eattr, Wh, wie)
        if it < DEPTH_ITERS - 1:
            h, p = _make_sc_fuse(True)(c, d, ei3)
        else:
            p = _make_sc_fuse(False)(c, d, ei3)
    return _k_final(p, vp, batch3, wov, wom, bo.reshape(1, DH), W1,
                    b1.reshape(1, EMB), W2, b2.reshape(1, EMB))
